# baseline (speedup 1.0000x reference)
import functools

import jax
import jax.numpy as jnp
from jax import lax
from jax.experimental import pallas as pl
from jax.experimental.pallas import tpu as pltpu
from jax.experimental.pallas import tpu_sc as plsc

N_NODES = 50000
D = 64
N_EDGES = 800000
NC, NS = 2, 16
NW = NC * NS
HALF = N_NODES // NC
RPT = 1568
PAD = RPT * NS
GAP = PAD - HALF
DUMP = HALF
NP2 = 2 * PAD

EPT = N_EDGES // NS
SCE = 2000
NSC = EPT // SCE
GC = 128
NCHMAX = 16
NSLOT = 2
TRASH = NCHMAX * GC

EPW = N_EDGES // NW
SCC = 196
LASTOFF = EPW - GC
NSS = 4

RB = NP2 // 8

_mesh = plsc.VectorSubcoreMesh(core_axis_name="c", subcore_axis_name="s")
_sc_params = pltpu.CompilerParams(needs_layout_passes=False,
                                  use_tc_tiling_on_sc=False)


def _pad_idx(v):
    return (v + jnp.where(v >= HALF, GAP, 0)).astype(jnp.int32)


def _flat2d(pos):
    return lax.shift_right_logical(pos, 7), pos & (GC - 1)


def _deg_body(col_hbm, z16_hbm, ones_hbm, deg_hbm, colb, cdst2, onesb, acc):
    c = lax.axis_index("c")
    s = lax.axis_index("s")
    base = c * HALF
    pltpu.sync_copy(z16_hbm, acc.at[pl.ds(s * RPT, RPT)])
    pltpu.sync_copy(ones_hbm, onesb)
    plsc.subcore_barrier()
    e0 = s * EPT

    def superchunk(sc_i, _):
        off = e0 + sc_i * SCE
        pltpu.sync_copy(col_hbm.at[pl.ds(off, SCE)], colb)

        def filt(gi, cnt):
            cv = colb[pl.ds(gi * 16, 16)]
            inr = (cv >= base) & (cv < base + HALF)
            loc = jnp.where(inr, cv - base, DUMP).astype(jnp.int32)
            cum = plsc.cumsum(inr.astype(jnp.int32))
            pos = jnp.where(inr, cnt + cum - 1,
                            TRASH + lax.iota(jnp.int32, 16))
            pr, pc = _flat2d(pos)
            plsc.store_scatter(cdst2, [pr, pc], loc)
            return cnt + cum[15]

        cnt = lax.fori_loop(0, SCE // 16, filt, jnp.int32(0))
        padv = DUMP + lax.iota(jnp.int32, 16)
        for k in range(GC // 16):
            pr, pc = _flat2d(cnt + k * 16 + lax.iota(jnp.int32, 16))
            plsc.store_scatter(cdst2, [pr, pc], padv)
        nch = (cnt + GC - 1) // GC
        for j in range(NCHMAX):
            @pl.when(j < nch)
            def _():
                pltpu.sync_copy(onesb, acc.at[cdst2.at[j]], add=True)
        return 0

    lax.fori_loop(0, NSC, superchunk, 0)
    plsc.subcore_barrier()
    pltpu.sync_copy(acc.at[pl.ds(s * RPT, RPT)],
                    deg_hbm.at[pl.ds(c * PAD + s * RPT, RPT)])


_deg_call = pl.kernel(
    _deg_body,
    out_type=jax.ShapeDtypeStruct((NP2, 16), jnp.float32),
    mesh=_mesh,
    scratch_types=[
        pltpu.VMEM((SCE,), jnp.int32),
        pltpu.VMEM((NCHMAX + 1, GC), jnp.int32),
        pltpu.VMEM((GC, 16), jnp.float32),
        pltpu.VMEM_SHARED((PAD, 16), jnp.float32),
    ],
    compiler_params=_sc_params,
)


def _prop_body(row_hbm, col_hbm, u_hbm, z64_hbm, s_hbm,
               rowb, colb, crow2, cdst2, stg, acc, gsem, isem):
    c = lax.axis_index("c")
    s = lax.axis_index("s")
    base = c * HALF
    pltpu.sync_copy(z64_hbm, acc.at[pl.ds(s * RPT, RPT)])
    plsc.subcore_barrier()
    e0 = s * EPT

    def _gath(j):
        return pltpu.make_async_copy(u_hbm.at[crow2.at[j]],
                                     stg.at[j % NSLOT], gsem.at[j])

    def _idx(sc_i, sl):
        off = e0 + sc_i * SCE
        return (pltpu.make_async_copy(row_hbm.at[pl.ds(off, SCE)],
                                      rowb.at[sl], isem.at[sl]),
                pltpu.make_async_copy(col_hbm.at[pl.ds(off, SCE)],
                                      colb.at[sl], isem.at[sl]))

    for cp in _idx(jnp.int32(0), 0):
        cp.start()

    def superchunk(sc_i, sl):
        for cp in _idx(sc_i, sl):
            cp.wait()

        @pl.when(sc_i + 1 < NSC)
        def _():
            for cp in _idx(sc_i + 1, 1 - sl):
                cp.start()

        def filt(gi, cnt):
            dsl = pl.ds(gi * 16, 16)
            r = rowb[sl, dsl]
            cv = colb[sl, dsl]
            rp = _pad_idx(r)
            inr = (cv >= base) & (cv < base + HALF)
            loc = jnp.where(inr, cv - base, DUMP).astype(jnp.int32)
            cum = plsc.cumsum(inr.astype(jnp.int32))
            pos = jnp.where(inr, cnt + cum - 1,
                            TRASH + lax.iota(jnp.int32, 16))
            pr, pc = _flat2d(pos)
            plsc.store_scatter(crow2, [pr, pc], rp)
            plsc.store_scatter(cdst2, [pr, pc], loc)
            return cnt + cum[15]

        cnt = lax.fori_loop(0, SCE // 16, filt, jnp.int32(0))
        padv = DUMP + lax.iota(jnp.int32, 16)
        zv = jnp.zeros((16,), jnp.int32)
        for k in range(GC // 16):
            pr, pc = _flat2d(cnt + k * 16 + lax.iota(jnp.int32, 16))
            plsc.store_scatter(cdst2, [pr, pc], padv)
            plsc.store_scatter(crow2, [pr, pc], zv)
        nch = (cnt + GC - 1) // GC

        @pl.when(0 < nch)
        def _():
            _gath(0).start()
        for j in range(NCHMAX):
            @pl.when(j < nch)
            def _():
                if j + 1 < NCHMAX:
                    @pl.when(j + 1 < nch)
                    def _():
                        _gath(j + 1).start()
                _gath(j).wait()
                pltpu.sync_copy(stg.at[j % NSLOT], acc.at[cdst2.at[j]],
                                add=True)
        return 1 - sl

    lax.fori_loop(0, NSC, superchunk, jnp.int32(0))
    plsc.subcore_barrier()
    pltpu.sync_copy(acc.at[pl.ds(s * RPT, RPT)],
                    s_hbm.at[pl.ds(c * PAD + s * RPT, RPT)])


_prop_call = pl.kernel(
    _prop_body,
    out_type=jax.ShapeDtypeStruct((NP2, D), jnp.float32),
    mesh=_mesh,
    scratch_types=[
        pltpu.VMEM((2, SCE), jnp.int32),
        pltpu.VMEM((2, SCE), jnp.int32),
        pltpu.VMEM((NCHMAX + 1, GC), jnp.int32),
        pltpu.VMEM((NCHMAX + 1, GC), jnp.int32),
        pltpu.VMEM((NSLOT, GC, D), jnp.float32),
        pltpu.VMEM_SHARED((PAD, D), jnp.float32),
        pltpu.SemaphoreType.DMA((NCHMAX,)),
        pltpu.SemaphoreType.DMA((2,)),
    ],
    compiler_params=_sc_params,
)


def _score_body(row_hbm, col_hbm, out_hbm, score_hbm,
                rowb2, colb2, stgs, stgd, scoreb2, isem, gsem, ssem):
    c = lax.axis_index("c")
    s = lax.axis_index("s")
    wid = c * NS + s
    e0 = wid * EPW

    def offs(i):
        return e0 + jnp.minimum(i * GC, LASTOFF)

    def idx_load(i, sl):
        pltpu.make_async_copy(row_hbm.at[pl.ds(offs(i), GC)], rowb2.at[sl],
                              isem.at[sl]).start()
        pltpu.make_async_copy(col_hbm.at[pl.ds(offs(i), GC)], colb2.at[sl],
                              isem.at[sl]).start()

    def idx_wait(i, sl):
        pltpu.make_async_copy(row_hbm.at[pl.ds(offs(i), GC)], rowb2.at[sl],
                              isem.at[sl]).wait()
        pltpu.make_async_copy(col_hbm.at[pl.ds(offs(i), GC)], colb2.at[sl],
                              isem.at[sl]).wait()

    def map_and_fire(i, sl):
        idx_wait(i, sl)
        for k in range(GC // 16):
            dsl = pl.ds(k * 16, 16)
            rowb2[sl, dsl] = _pad_idx(rowb2[sl, dsl])
            colb2[sl, dsl] = _pad_idx(colb2[sl, dsl])
        pltpu.make_async_copy(out_hbm.at[rowb2.at[sl]], stgs.at[sl],
                              gsem.at[sl]).start()
        pltpu.make_async_copy(out_hbm.at[colb2.at[sl]], stgd.at[sl],
                              gsem.at[sl]).start()

    def gath_wait(sl):
        pltpu.make_async_copy(out_hbm.at[rowb2.at[sl]], stgs.at[sl],
                              gsem.at[sl]).wait()
        pltpu.make_async_copy(out_hbm.at[colb2.at[sl]], stgd.at[sl],
                              gsem.at[sl]).wait()

    def store_wait(i, sl):
        pltpu.make_async_copy(scoreb2.at[sl],
                              score_hbm.at[pl.ds(offs(i), GC)],
                              ssem.at[sl]).wait()

    for k in range(NSS):
        idx_load(jnp.int32(k), k)
    for k in range(2):
        map_and_fire(jnp.int32(k), k)

    def block(t, _):
        for k in range(NSS):
            i = t * NSS + k
            gath_wait(k)

            @pl.when(i >= NSS)
            def _():
                store_wait(i - NSS, k)

            def group(g, _):
                lane = lax.iota(jnp.int32, 16)
                es = lane + g * 16
                acc = jnp.zeros((16,), jnp.float32)
                for d in range(D):
                    dd = (lane + d) & (D - 1)
                    va = plsc.load_gather(stgs.at[k], [es, dd])
                    vb = plsc.load_gather(stgd.at[k], [es, dd])
                    acc = acc + va * vb
                scoreb2[k, pl.ds(g * 16, 16)] = acc
                return 0

            lax.fori_loop(0, GC // 16, group, 0)
            pltpu.make_async_copy(scoreb2.at[k],
                                  score_hbm.at[pl.ds(offs(i), GC)],
                                  ssem.at[k]).start()

            @pl.when(i + 2 < SCC)
            def _():
                map_and_fire(i + 2, (k + 2) % NSS)

            @pl.when(i + NSS < SCC)
            def _():
                idx_load(i + NSS, k)
        return 0

    lax.fori_loop(0, SCC // NSS, block, 0)
    for k in range(NSS):
        store_wait(jnp.int32(SCC - NSS + k), k)


_score_call = pl.kernel(
    _score_body,
    out_type=jax.ShapeDtypeStruct((N_EDGES,), jnp.float32),
    mesh=_mesh,
    scratch_types=[
        pltpu.VMEM((NSS, GC), jnp.int32),
        pltpu.VMEM((NSS, GC), jnp.int32),
        pltpu.VMEM((NSS, GC, D), jnp.float32),
        pltpu.VMEM((NSS, GC, D), jnp.float32),
        pltpu.VMEM((NSS, GC), jnp.float32),
        pltpu.SemaphoreType.DMA((NSS,)),
        pltpu.SemaphoreType.DMA((NSS,)),
        pltpu.SemaphoreType.DMA((NSS,)),
    ],
    compiler_params=_sc_params,
)


def _prep_body(deg16_ref, embp_ref, u0_ref, d2_ref, g_ref):
    i = pl.program_id(0)
    deg = deg16_ref[:, 0:1]
    ridx = lax.broadcasted_iota(jnp.int32, (RB, 1), 0) + i * RB
    valid = (ridx % PAD) < HALF
    dpos = valid & (deg > 0)
    dis = jnp.where(dpos, lax.rsqrt(jnp.maximum(deg, 1.0)), 0.0)
    d2_ref[...] = dis * dis
    g_ref[...] = deg * dis
    u0_ref[...] = embp_ref[...] * dis


def _scale_body(s_ref, d2_ref, u_ref):
    u_ref[...] = s_ref[...] * d2_ref[...]


def _blend_body(embp_ref, g_ref, d2_ref, u1_ref, u2_ref, s3_ref, alpha_ref,
                out_ref):
    a0 = alpha_ref[0]
    a1 = alpha_ref[1]
    a2 = alpha_ref[2]
    a3 = alpha_ref[3]
    out_ref[...] = a0 * embp_ref[...] + g_ref[...] * (
        a1 * u1_ref[...] + a2 * u2_ref[...] + a3 * (d2_ref[...] * s3_ref[...]))


def _rows(w):
    return pl.BlockSpec((RB, w), lambda i: (i, 0))


_prep_call = pl.pallas_call(
    _prep_body,
    grid=(8,),
    in_specs=[_rows(16), _rows(D)],
    out_specs=(_rows(D), _rows(1), _rows(1)),
    out_shape=(jax.ShapeDtypeStruct((NP2, D), jnp.float32),
               jax.ShapeDtypeStruct((NP2, 1), jnp.float32),
               jax.ShapeDtypeStruct((NP2, 1), jnp.float32)),
)

_scale_call = pl.pallas_call(
    _scale_body,
    grid=(8,),
    in_specs=[_rows(D), _rows(1)],
    out_specs=_rows(D),
    out_shape=jax.ShapeDtypeStruct((NP2, D), jnp.float32),
)

_blend_call = pl.pallas_call(
    _blend_body,
    grid=(8,),
    in_specs=[_rows(D), _rows(1), _rows(1), _rows(D), _rows(D), _rows(D),
              pl.BlockSpec(memory_space=pltpu.SMEM)],
    out_specs=_rows(D),
    out_shape=jax.ShapeDtypeStruct((NP2, D), jnp.float32),
)


@jax.jit
def kernel(edge_index, embedding, alpha):
    row = edge_index[0]
    col = edge_index[1]
    zrow = jnp.zeros((GAP, D), jnp.float32)
    embp = jnp.concatenate(
        [embedding[:HALF], zrow, embedding[HALF:], zrow], axis=0)
    z16 = jnp.zeros((RPT, 16), jnp.float32)
    z64 = jnp.zeros((RPT, D), jnp.float32)
    ones16 = jnp.ones((GC, 16), jnp.float32)
    alpha = alpha.astype(jnp.float32)

    deg16 = _deg_call(col, z16, ones16)
    u0, d2, g = _prep_call(deg16, embp)
    s1 = _prop_call(row, col, u0, z64)
    u1 = _scale_call(s1, d2)
    s2 = _prop_call(row, col, u1, z64)
    u2 = _scale_call(s2, d2)
    s3 = _prop_call(row, col, u2, z64)
    out = _blend_call(embp, g, d2, u1, u2, s3, alpha)
    score = _score_call(row, col, out)
    return score
